# baseline (speedup 1.0000x reference)
import functools

import jax
import jax.numpy as jnp
from jax import lax
from jax.experimental import pallas as pl
from jax.experimental.pallas import tpu as pltpu
from jax.experimental.pallas import tpu_sc as plsc

_LANES = 16
_NUM_CORES = 2
_NUM_SUBCORES = 16
_CHUNK = 8
_NBUF = 4


def _build_combine(T, H, K, C):
    NW = _NUM_CORES * _NUM_SUBCORES
    NB = _NBUF
    tok_per_w = T // NW
    nchunk = tok_per_w // C
    HV = H // _LANES
    PADW = K * C + _LANES
    mesh = plsc.VectorSubcoreMesh(core_axis_name="c", subcore_axis_name="s")

    @functools.partial(
        pl.kernel,
        out_type=jax.ShapeDtypeStruct((T, H), jnp.float32),
        mesh=mesh,
        scratch_types=[
            pltpu.VMEM((nchunk, K * C), jnp.int32),
            pltpu.VMEM((nchunk, PADW), jnp.float32),
            [pltpu.VMEM((K * C, H), jnp.float32)] * NB,
            [pltpu.VMEM((C, H), jnp.float32)] * 2,
            [pltpu.SemaphoreType.DMA] * NB,
            [pltpu.SemaphoreType.DMA] * 2,
        ],
    )
    def combine(table_hbm, idx_hbm, w_hbm, out_hbm, idx_v, w_v,
                rows, outs, sg, ss):
        wid = lax.axis_index("s") * _NUM_CORES + lax.axis_index("c")
        base = wid * tok_per_w

        pltpu.sync_copy(idx_hbm.at[wid], idx_v)
        pltpu.sync_copy(w_hbm.at[wid], w_v)

        def gather(j, p):
            return pltpu.make_async_copy(
                table_hbm.at[idx_v.at[j]], rows[p], sg[p])

        def scatter(j, q):
            return pltpu.make_async_copy(
                outs[q], out_hbm.at[pl.ds(base + j * C, C)], ss[q])

        for p in range(NB):
            gather(p, p).start()

        def ring_body(jj, _):
            for p in range(NB):
                j = jj * NB + p
                q = p % 2
                gather(j, p).wait()

                @pl.when(j >= 2)
                def _wait_prev_scatter():
                    scatter(j - 2, q).wait()

                rbuf = rows[p]
                obuf = outs[q]

                @plsc.parallel_loop(0, C, step=1, unroll=4)
                def per_token(c):
                    w16 = w_v[j, pl.ds(K * c, _LANES)]
                    w0 = w16[0]
                    w1 = w16[1]
                    for h in range(HV):
                        hs = pl.ds(h * _LANES, _LANES)
                        obuf[c, hs] = (w0 * rbuf[K * c, hs]
                                       + w1 * rbuf[K * c + 1, hs])

                scatter(j, q).start()

                @pl.when(j + NB < nchunk)
                def _prefetch_gather():
                    gather(j + NB, p).start()
            return 0

        lax.fori_loop(0, nchunk // NB, ring_body, 0)
        scatter(nchunk - 2, (nchunk - 2) % 2).wait()
        scatter(nchunk - 1, (nchunk - 1) % 2).wait()

    return combine


def kernel(hidden_states, expert_indices, expert_weights, expert_outputs):
    B, S, H = hidden_states.shape
    E = expert_outputs.shape[0]
    K = expert_indices.shape[-1]
    T = B * S
    C = _CHUNK
    NW = _NUM_CORES * _NUM_SUBCORES
    nchunk = T // (NW * C)
    table = expert_outputs.reshape(E * T, H).astype(jnp.float32)
    tok = jnp.arange(T, dtype=jnp.int32)[:, None]
    row_idx = (expert_indices.reshape(T, K).astype(jnp.int32) * T
               + tok).reshape(NW, nchunk, K * C)
    w = expert_weights.reshape(NW, nchunk, K * C).astype(jnp.float32)
    w = jnp.pad(w, ((0, 0), (0, 0), (0, _LANES)))
    out = _build_combine(T, H, K, C)(table, row_idx, w)
    return out.reshape(B, S, H).astype(hidden_states.dtype)


# speedup vs baseline: 1.6224x; 1.6224x over previous
import functools

import jax
import jax.numpy as jnp
from jax import lax
from jax.experimental import pallas as pl
from jax.experimental.pallas import tpu as pltpu
from jax.experimental.pallas import tpu_sc as plsc

_LANES = 16
_NUM_CORES = 2
_NUM_SUBCORES = 16
_CHUNK = 16
_NBUF = 3


def _build_combine(T, H, K, C):
    NW = _NUM_CORES * _NUM_SUBCORES
    NB = _NBUF
    tok_per_w = T // NW
    nchunk = tok_per_w // C
    HV = H // _LANES
    PADW = K * C + _LANES
    mesh = plsc.VectorSubcoreMesh(core_axis_name="c", subcore_axis_name="s")

    @functools.partial(
        pl.kernel,
        out_type=jax.ShapeDtypeStruct((T, H), jnp.float32),
        mesh=mesh,
        scratch_types=[
            pltpu.VMEM((nchunk, K * C), jnp.int32),
            pltpu.VMEM((nchunk, PADW), jnp.float32),
            [pltpu.VMEM((K * C, H), jnp.float32)] * NB,
            [pltpu.SemaphoreType.DMA] * NB,
            [pltpu.SemaphoreType.DMA] * NB,
        ],
    )
    def combine(table_hbm, idx_hbm, w_hbm, out_hbm, idx_v, w_v,
                rows, sg, ss):
        wid = lax.axis_index("s") * _NUM_CORES + lax.axis_index("c")
        base = wid * tok_per_w

        pltpu.sync_copy(idx_hbm.at[wid], idx_v)
        pltpu.sync_copy(w_hbm.at[wid], w_v)

        def gather(j, p):
            return pltpu.make_async_copy(
                table_hbm.at[idx_v.at[j]], rows[p], sg[p])

        def scatter(j, p):
            return pltpu.make_async_copy(
                rows[p].at[pl.ds(0, C)],
                out_hbm.at[pl.ds(base + j * C, C)], ss[p])

        for p in range(NB):
            gather(p, p).start()

        def chunk_step(j, p):
            gather(j, p).wait()
            rbuf = rows[p]

            def per_token(c, _):
                wa = w_v[j, pl.ds(c, _LANES)]
                wb = w_v[j, pl.ds(C + c, _LANES)]
                w0 = wa[0]
                w1 = wb[0]

                @plsc.parallel_loop(0, HV, step=1, unroll=8)
                def per_h(h):
                    hs = pl.ds(h * _LANES, _LANES)
                    rbuf[c, hs] = w0 * rbuf[c, hs] + w1 * rbuf[C + c, hs]

                return 0

            lax.fori_loop(0, C, per_token, 0)
            scatter(j, p).start()

            @pl.when(j + NB < nchunk)
            def _refill():
                scatter(j, p).wait()
                gather(j + NB, p).start()

        def ring_body(jj, _):
            for p in range(NB):
                chunk_step(jj * NB + p, p)
            return 0

        nfull = nchunk // NB
        lax.fori_loop(0, nfull, ring_body, 0)
        for j in range(nfull * NB, nchunk):
            chunk_step(j, j % NB)
        for j in range(max(0, nchunk - NB), nchunk):
            scatter(j, j % NB).wait()

    return combine


def kernel(hidden_states, expert_indices, expert_weights, expert_outputs):
    B, S, H = hidden_states.shape
    E = expert_outputs.shape[0]
    K = expert_indices.shape[-1]
    T = B * S
    C = _CHUNK
    NW = _NUM_CORES * _NUM_SUBCORES
    nchunk = T // (NW * C)
    table = expert_outputs.reshape(E * T, H).astype(jnp.float32)
    tok = jnp.arange(T, dtype=jnp.int32)[:, None]
    row_idx = (expert_indices.reshape(T, K).astype(jnp.int32) * T + tok)
    row_idx = (row_idx.reshape(NW, nchunk, C, K).swapaxes(2, 3)
               .reshape(NW, nchunk, K * C))
    w = (expert_weights.reshape(NW, nchunk, C, K).astype(jnp.float32)
         .swapaxes(2, 3).reshape(NW, nchunk, K * C))
    w = jnp.pad(w, ((0, 0), (0, 0), (0, _LANES)))
    out = _build_combine(T, H, K, C)(table, row_idx, w)
    return out.reshape(B, S, H).astype(hidden_states.dtype)


# speedup vs baseline: 1.6330x; 1.0065x over previous
import functools

import jax
import jax.numpy as jnp
from jax import lax
from jax.experimental import pallas as pl
from jax.experimental.pallas import tpu as pltpu
from jax.experimental.pallas import tpu_sc as plsc

_LANES = 16
_NUM_CORES = 2
_NUM_SUBCORES = 16
_CHUNK = 8
_NBUF = 4


def _build_combine(T, H, K, C):
    NW = _NUM_CORES * _NUM_SUBCORES
    NB = _NBUF
    tok_per_w = T // NW
    nchunk = tok_per_w // C
    HV = H // _LANES
    PADW = K * C + _LANES
    mesh = plsc.VectorSubcoreMesh(core_axis_name="c", subcore_axis_name="s")

    @functools.partial(
        pl.kernel,
        out_type=jax.ShapeDtypeStruct((T, H), jnp.float32),
        mesh=mesh,
        scratch_types=[
            pltpu.VMEM((nchunk, K * C), jnp.int32),
            pltpu.VMEM((nchunk, PADW), jnp.float32),
            [pltpu.VMEM((K * C, H), jnp.float32)] * NB,
            [pltpu.SemaphoreType.DMA] * NB,
            [pltpu.SemaphoreType.DMA] * NB,
        ],
    )
    def combine(table_hbm, idx_hbm, w_hbm, out_hbm, idx_v, w_v,
                rows, sg, ss):
        wid = lax.axis_index("s") * _NUM_CORES + lax.axis_index("c")
        base = wid * tok_per_w

        pltpu.sync_copy(idx_hbm.at[wid], idx_v)
        pltpu.sync_copy(w_hbm.at[wid], w_v)

        def gather(j, p):
            return pltpu.make_async_copy(
                table_hbm.at[idx_v.at[j]], rows[p], sg[p])

        def scatter(j, p):
            return pltpu.make_async_copy(
                rows[p].at[pl.ds(0, C)],
                out_hbm.at[pl.ds(base + j * C, C)], ss[p])

        for p in range(NB):
            gather(p, p).start()

        def chunk_step(j, p):
            gather(j, p).wait()
            rbuf = rows[p]

            def per_token(c, _):
                wa = w_v[j, pl.ds(c, _LANES)]
                wb = w_v[j, pl.ds(C + c, _LANES)]
                w0 = wa[0]
                w1 = wb[0]

                @plsc.parallel_loop(0, HV, step=1, unroll=8)
                def per_h(h):
                    hs = pl.ds(h * _LANES, _LANES)
                    rbuf[c, hs] = w0 * rbuf[c, hs] + w1 * rbuf[C + c, hs]

                return 0

            lax.fori_loop(0, C, per_token, 0)
            scatter(j, p).start()

            @pl.when(j + NB < nchunk)
            def _refill():
                scatter(j, p).wait()
                gather(j + NB, p).start()

        def ring_body(jj, _):
            for p in range(NB):
                chunk_step(jj * NB + p, p)
            return 0

        nfull = nchunk // NB
        lax.fori_loop(0, nfull, ring_body, 0)
        for j in range(nfull * NB, nchunk):
            chunk_step(j, j % NB)
        for j in range(max(0, nchunk - NB), nchunk):
            scatter(j, j % NB).wait()

    return combine


def kernel(hidden_states, expert_indices, expert_weights, expert_outputs):
    B, S, H = hidden_states.shape
    E = expert_outputs.shape[0]
    K = expert_indices.shape[-1]
    T = B * S
    C = _CHUNK
    NW = _NUM_CORES * _NUM_SUBCORES
    nchunk = T // (NW * C)
    table = expert_outputs.reshape(E * T, H).astype(jnp.float32)
    tok = jnp.arange(T, dtype=jnp.int32)[:, None]
    row_idx = (expert_indices.reshape(T, K).astype(jnp.int32) * T + tok)
    row_idx = (row_idx.reshape(NW, nchunk, C, K).swapaxes(2, 3)
               .reshape(NW, nchunk, K * C))
    w = (expert_weights.reshape(NW, nchunk, C, K).astype(jnp.float32)
         .swapaxes(2, 3).reshape(NW, nchunk, K * C))
    w = jnp.pad(w, ((0, 0), (0, 0), (0, _LANES)))
    out = _build_combine(T, H, K, C)(table, row_idx, w)
    return out.reshape(B, S, H).astype(hidden_states.dtype)


# speedup vs baseline: 1.6551x; 1.0135x over previous
import functools

import jax
import jax.numpy as jnp
from jax import lax
from jax.experimental import pallas as pl
from jax.experimental.pallas import tpu as pltpu
from jax.experimental.pallas import tpu_sc as plsc

_LANES = 16
_NUM_CORES = 2
_NUM_SUBCORES = 16
_CHUNK = 8
_NBUF = 6


def _build_combine(T, H, K, C):
    NW = _NUM_CORES * _NUM_SUBCORES
    NB = _NBUF
    tok_per_w = T // NW
    nchunk = tok_per_w // C
    HV = H // _LANES
    PADW = K * C + _LANES
    mesh = plsc.VectorSubcoreMesh(core_axis_name="c", subcore_axis_name="s")

    @functools.partial(
        pl.kernel,
        out_type=jax.ShapeDtypeStruct((T, H), jnp.float32),
        mesh=mesh,
        scratch_types=[
            pltpu.VMEM((nchunk, K * C), jnp.int32),
            pltpu.VMEM((nchunk, PADW), jnp.float32),
            [pltpu.VMEM((K * C, H), jnp.float32)] * NB,
            [pltpu.SemaphoreType.DMA] * NB,
            [pltpu.SemaphoreType.DMA] * NB,
        ],
    )
    def combine(table_hbm, idx_hbm, w_hbm, out_hbm, idx_v, w_v,
                rows, sg, ss):
        wid = lax.axis_index("s") * _NUM_CORES + lax.axis_index("c")
        base = wid * tok_per_w

        pltpu.sync_copy(idx_hbm.at[wid], idx_v)
        pltpu.sync_copy(w_hbm.at[wid], w_v)

        def gather(j, p):
            return pltpu.make_async_copy(
                table_hbm.at[idx_v.at[j]], rows[p], sg[p])

        def scatter(j, p):
            return pltpu.make_async_copy(
                rows[p].at[pl.ds(0, C)],
                out_hbm.at[pl.ds(base + j * C, C)], ss[p])

        for p in range(NB):
            gather(p, p).start()

        def chunk_step(j, p):
            gather(j, p).wait()
            rbuf = rows[p]

            def per_token(c, _):
                wa = w_v[j, pl.ds(c, _LANES)]
                wb = w_v[j, pl.ds(C + c, _LANES)]
                w0 = wa[0]
                w1 = wb[0]

                @plsc.parallel_loop(0, HV, step=1, unroll=8)
                def per_h(h):
                    hs = pl.ds(h * _LANES, _LANES)
                    rbuf[c, hs] = w0 * rbuf[c, hs] + w1 * rbuf[C + c, hs]

                return 0

            lax.fori_loop(0, C, per_token, 0)
            scatter(j, p).start()

            @pl.when(j + NB < nchunk)
            def _refill():
                scatter(j, p).wait()
                gather(j + NB, p).start()

        def ring_body(jj, _):
            for p in range(NB):
                chunk_step(jj * NB + p, p)
            return 0

        nfull = nchunk // NB
        lax.fori_loop(0, nfull, ring_body, 0)
        for j in range(nfull * NB, nchunk):
            chunk_step(j, j % NB)
        for j in range(max(0, nchunk - NB), nchunk):
            scatter(j, j % NB).wait()

    return combine


def kernel(hidden_states, expert_indices, expert_weights, expert_outputs):
    B, S, H = hidden_states.shape
    E = expert_outputs.shape[0]
    K = expert_indices.shape[-1]
    T = B * S
    C = _CHUNK
    NW = _NUM_CORES * _NUM_SUBCORES
    nchunk = T // (NW * C)
    table = expert_outputs.reshape(E * T, H).astype(jnp.float32)
    tok = jnp.arange(T, dtype=jnp.int32)[:, None]
    row_idx = (expert_indices.reshape(T, K).astype(jnp.int32) * T + tok)
    row_idx = (row_idx.reshape(NW, nchunk, C, K).swapaxes(2, 3)
               .reshape(NW, nchunk, K * C))
    w = (expert_weights.reshape(NW, nchunk, C, K).astype(jnp.float32)
         .swapaxes(2, 3).reshape(NW, nchunk, K * C))
    w = jnp.pad(w, ((0, 0), (0, 0), (0, _LANES)))
    out = _build_combine(T, H, K, C)(table, row_idx, w)
    return out.reshape(B, S, H).astype(hidden_states.dtype)


# speedup vs baseline: 1.6620x; 1.0042x over previous
import functools

import jax
import jax.numpy as jnp
from jax import lax
from jax.experimental import pallas as pl
from jax.experimental.pallas import tpu as pltpu
from jax.experimental.pallas import tpu_sc as plsc

_LANES = 16
_NUM_CORES = 2
_NUM_SUBCORES = 16
_CHUNK = 8
_NBUF = 7


def _build_combine(T, H, K, C):
    NW = _NUM_CORES * _NUM_SUBCORES
    NB = _NBUF
    tok_per_w = T // NW
    nchunk = tok_per_w // C
    HV = H // _LANES
    PADW = K * C + _LANES
    mesh = plsc.VectorSubcoreMesh(core_axis_name="c", subcore_axis_name="s")

    @functools.partial(
        pl.kernel,
        out_type=jax.ShapeDtypeStruct((T, H), jnp.float32),
        mesh=mesh,
        scratch_types=[
            pltpu.VMEM((nchunk, K * C), jnp.int32),
            pltpu.VMEM((nchunk, PADW), jnp.float32),
            [pltpu.VMEM((K * C, H), jnp.float32)] * NB,
            [pltpu.SemaphoreType.DMA] * NB,
            [pltpu.SemaphoreType.DMA] * NB,
        ],
    )
    def combine(table_hbm, idx_hbm, w_hbm, out_hbm, idx_v, w_v,
                rows, sg, ss):
        wid = lax.axis_index("s") * _NUM_CORES + lax.axis_index("c")
        base = wid * tok_per_w

        pltpu.sync_copy(idx_hbm.at[wid], idx_v)
        pltpu.sync_copy(w_hbm.at[wid], w_v)

        def gather(j, p):
            return pltpu.make_async_copy(
                table_hbm.at[idx_v.at[j]], rows[p], sg[p])

        def scatter(j, p):
            return pltpu.make_async_copy(
                rows[p].at[pl.ds(0, C)],
                out_hbm.at[pl.ds(base + j * C, C)], ss[p])

        for p in range(NB):
            gather(p, p).start()

        def chunk_step(j, p):
            gather(j, p).wait()
            rbuf = rows[p]

            def per_token(c, _):
                wa = w_v[j, pl.ds(c, _LANES)]
                wb = w_v[j, pl.ds(C + c, _LANES)]
                w0 = wa[0]
                w1 = wb[0]

                @plsc.parallel_loop(0, HV, step=1, unroll=8)
                def per_h(h):
                    hs = pl.ds(h * _LANES, _LANES)
                    rbuf[c, hs] = w0 * rbuf[c, hs] + w1 * rbuf[C + c, hs]

                return 0

            lax.fori_loop(0, C, per_token, 0)
            scatter(j, p).start()

            @pl.when(j + NB < nchunk)
            def _refill():
                scatter(j, p).wait()
                gather(j + NB, p).start()

        def ring_body(jj, _):
            for p in range(NB):
                chunk_step(jj * NB + p, p)
            return 0

        nfull = nchunk // NB
        lax.fori_loop(0, nfull, ring_body, 0)
        for j in range(nfull * NB, nchunk):
            chunk_step(j, j % NB)
        for j in range(max(0, nchunk - NB), nchunk):
            scatter(j, j % NB).wait()

    return combine


def kernel(hidden_states, expert_indices, expert_weights, expert_outputs):
    B, S, H = hidden_states.shape
    E = expert_outputs.shape[0]
    K = expert_indices.shape[-1]
    T = B * S
    C = _CHUNK
    NW = _NUM_CORES * _NUM_SUBCORES
    nchunk = T // (NW * C)
    table = expert_outputs.reshape(E * T, H).astype(jnp.float32)
    tok = jnp.arange(T, dtype=jnp.int32)[:, None]
    row_idx = (expert_indices.reshape(T, K).astype(jnp.int32) * T + tok)
    row_idx = (row_idx.reshape(NW, nchunk, C, K).swapaxes(2, 3)
               .reshape(NW, nchunk, K * C))
    w = (expert_weights.reshape(NW, nchunk, C, K).astype(jnp.float32)
         .swapaxes(2, 3).reshape(NW, nchunk, K * C))
    w = jnp.pad(w, ((0, 0), (0, 0), (0, _LANES)))
    out = _build_combine(T, H, K, C)(table, row_idx, w)
    return out.reshape(B, S, H).astype(hidden_states.dtype)


# speedup vs baseline: 1.6696x; 1.0046x over previous
import functools

import jax
import jax.numpy as jnp
from jax import lax
from jax.experimental import pallas as pl
from jax.experimental.pallas import tpu as pltpu
from jax.experimental.pallas import tpu_sc as plsc

_LANES = 16
_NUM_CORES = 2
_NUM_SUBCORES = 16
_CHUNK = 8
_NBUF = 7


def _build_combine(T, H, K, C):
    NW = _NUM_CORES * _NUM_SUBCORES
    NB = _NBUF
    tok_per_w = T // NW
    nchunk = tok_per_w // C
    HV = H // _LANES
    PADW = K * C + _LANES
    mesh = plsc.VectorSubcoreMesh(core_axis_name="c", subcore_axis_name="s")

    @functools.partial(
        pl.kernel,
        out_type=jax.ShapeDtypeStruct((T, H), jnp.float32),
        mesh=mesh,
        scratch_types=[
            pltpu.VMEM((nchunk, K * C), jnp.int32),
            pltpu.VMEM((nchunk, PADW), jnp.float32),
            [pltpu.VMEM((K * C, H), jnp.float32)] * NB,
            [pltpu.SemaphoreType.DMA] * NB,
            [pltpu.SemaphoreType.DMA] * NB,
        ],
    )
    def combine(table_hbm, idx_hbm, w_hbm, out_hbm, idx_v, w_v,
                rows, sg, ss):
        wid = lax.axis_index("s") * _NUM_CORES + lax.axis_index("c")
        base = wid * tok_per_w

        pltpu.sync_copy(idx_hbm.at[wid], idx_v)

        def gather(j, p):
            return pltpu.make_async_copy(
                table_hbm.at[idx_v.at[j]], rows[p], sg[p])

        def scatter(j, p):
            return pltpu.make_async_copy(
                rows[p].at[pl.ds(0, C)],
                out_hbm.at[pl.ds(base + j * C, C)], ss[p])

        for p in range(NB):
            gather(p, p).start()
        pltpu.sync_copy(w_hbm.at[wid], w_v)

        def chunk_step(j, p):
            gather(j, p).wait()
            rbuf = rows[p]

            def per_token(c, _):
                wa = w_v[j, pl.ds(c, _LANES)]
                wb = w_v[j, pl.ds(C + c, _LANES)]
                w0 = wa[0]
                w1 = wb[0]

                @plsc.parallel_loop(0, HV, step=1, unroll=8)
                def per_h(h):
                    hs = pl.ds(h * _LANES, _LANES)
                    rbuf[c, hs] = w0 * rbuf[c, hs] + w1 * rbuf[C + c, hs]

                return 0

            lax.fori_loop(0, C, per_token, 0)
            scatter(j, p).start()

            @pl.when(j + NB < nchunk)
            def _refill():
                scatter(j, p).wait()
                gather(j + NB, p).start()

        def ring_body(jj, _):
            for p in range(NB):
                chunk_step(jj * NB + p, p)
            return 0

        nfull = nchunk // NB
        lax.fori_loop(0, nfull, ring_body, 0)
        for j in range(nfull * NB, nchunk):
            chunk_step(j, j % NB)
        for j in range(max(0, nchunk - NB), nchunk):
            scatter(j, j % NB).wait()

    return combine


def kernel(hidden_states, expert_indices, expert_weights, expert_outputs):
    B, S, H = hidden_states.shape
    E = expert_outputs.shape[0]
    K = expert_indices.shape[-1]
    T = B * S
    C = _CHUNK
    NW = _NUM_CORES * _NUM_SUBCORES
    nchunk = T // (NW * C)
    table = expert_outputs.reshape(E * T, H).astype(jnp.float32)
    tok = jnp.arange(T, dtype=jnp.int32)[:, None]
    row_idx = (expert_indices.reshape(T, K).astype(jnp.int32) * T + tok)
    row_idx = (row_idx.reshape(NW, nchunk, C, K).swapaxes(2, 3)
               .reshape(NW, nchunk, K * C))
    w = (expert_weights.reshape(NW, nchunk, C, K).astype(jnp.float32)
         .swapaxes(2, 3).reshape(NW, nchunk, K * C))
    w = jnp.pad(w, ((0, 0), (0, 0), (0, _LANES)))
    out = _build_combine(T, H, K, C)(table, row_idx, w)
    return out.reshape(B, S, H).astype(hidden_states.dtype)
